# Initial kernel scaffold; baseline (speedup 1.0000x reference)
import jax
import jax.numpy as jnp
from jax.experimental import pallas as pl


def kernel(inputs, table):
    raise NotImplementedError("write your pallas kernel here")


# speedup vs baseline: 1.0938x; 1.0938x over previous
import functools

import jax
import jax.numpy as jnp
from jax import lax
from jax.experimental import pallas as pl
from jax.experimental.pallas import tpu as pltpu
from jax.experimental.pallas import tpu_sc as plsc

NC = 2
NS = 16
NW = NC * NS

CHUNK = 1024
SUB = 128
N_SUB = CHUNK // SUB


def _make_gather(B, V, D):
    b_per_w = B // NW
    n_chunk = b_per_w // CHUNK
    mesh = plsc.VectorSubcoreMesh(
        core_axis_name="c", subcore_axis_name="s", num_cores=NC, num_subcores=NS)

    @functools.partial(
        pl.kernel,
        out_type=jax.ShapeDtypeStruct((B, D), jnp.float32),
        mesh=mesh,
        scratch_types=[
            pltpu.VMEM((CHUNK,), jnp.int32),
            pltpu.VMEM((CHUNK, D), jnp.float32),
            pltpu.SemaphoreType.DMA,
        ],
        compiler_params=pltpu.CompilerParams(use_tc_tiling_on_sc=False),
    )
    def gather_kernel(idx_hbm, table_hbm, out_hbm, idx_v, rows_v, sem):
        wid = lax.axis_index("s") * NC + lax.axis_index("c")
        base = wid * b_per_w

        def step(i):
            off = base + i * CHUNK
            pltpu.sync_copy(idx_hbm.at[pl.ds(off, CHUNK)], idx_v)
            descs = []
            for j in range(N_SUB):
                descs.append(pltpu.async_copy(
                    table_hbm.at[idx_v.at[pl.ds(j * SUB, SUB)]],
                    rows_v.at[pl.ds(j * SUB, SUB)],
                    sem))
            for d in descs:
                d.wait()
            pltpu.sync_copy(rows_v, out_hbm.at[pl.ds(off, CHUNK)])

        pl.loop(0, n_chunk)(step)

    return gather_kernel


def kernel(inputs, table):
    BATCH, HIST = inputs.shape
    V, D = table.shape
    B = BATCH * HIST
    idx = inputs.reshape(B).astype(jnp.int32)
    flat = _make_gather(B, V, D)(idx, table)
    return flat.reshape(BATCH, HIST, D)


# speedup vs baseline: 1.7395x; 1.5903x over previous
import functools

import jax
import jax.numpy as jnp
from jax import lax
from jax.experimental import pallas as pl
from jax.experimental.pallas import tpu as pltpu
from jax.experimental.pallas import tpu_sc as plsc

NC = 2
NS = 16
NW = NC * NS

NB = 16


def _make_gather(BATCH, HIST, V, D):
    b_per_w = BATCH // NW
    n_chunk = b_per_w // NB
    mesh = plsc.VectorSubcoreMesh(
        core_axis_name="c", subcore_axis_name="s", num_cores=NC, num_subcores=NS)

    @functools.partial(
        pl.kernel,
        out_type=jax.ShapeDtypeStruct((BATCH, HIST, D), jnp.float32),
        mesh=mesh,
        scratch_types=[
            pltpu.VMEM((NB, HIST), jnp.int32),
            pltpu.VMEM((NB, HIST, D), jnp.float32),
            pltpu.SemaphoreType.DMA,
        ],
        compiler_params=pltpu.CompilerParams(use_tc_tiling_on_sc=False),
    )
    def gather_kernel(idx_hbm, table_hbm, out_hbm, idx_v, rows_v, sem):
        wid = lax.axis_index("s") * NC + lax.axis_index("c")
        base = wid * b_per_w

        def step(i):
            b0 = base + i * NB
            pltpu.sync_copy(idx_hbm.at[pl.ds(b0, NB)], idx_v)
            descs = []
            for r in range(NB):
                descs.append(pltpu.async_copy(
                    table_hbm.at[idx_v.at[r]], rows_v.at[r], sem))
            for d in descs:
                d.wait()
            pltpu.sync_copy(rows_v, out_hbm.at[pl.ds(b0, NB)])

        pl.loop(0, n_chunk)(step)

    return gather_kernel


def kernel(inputs, table):
    BATCH, HIST = inputs.shape
    V, D = table.shape
    return _make_gather(BATCH, HIST, V, D)(inputs.astype(jnp.int32), table)
